# baseline (speedup 1.0000x reference)
import functools

import jax
import jax.numpy as jnp
from jax import lax
from jax.experimental import pallas as pl
from jax.experimental.pallas import tpu as pltpu
from jax.experimental.pallas import tpu_sc as plsc

N_NODES = 50000
DIM = 128
BATCH = 4
K = max(1, int(N_NODES * 0.1))
ROW_BLK = 10000
NBLK = N_NODES // ROW_BLK

WPB = 8
W_ROWS = 6256
W_VB = W_ROWS // 16
SCORES_PAD = 64
CAP = 5120
CHUNK = 256

P1_BLK = 25000


def _scores_kernel(x_ref, w_ref, s_ref):
    prod = x_ref[...] * w_ref[...]
    s_ref[0] = jnp.sum(prod, axis=1).reshape(1, P1_BLK)


def _stats_kernel(s_full_ref, params_ref, loss_ref):
    b = pl.program_id(0)
    nf = float(N_NODES)
    kf = float(K)
    eps = 1e-8

    S = s_full_ref[0]
    mean = jnp.sum(S) / nf
    var = jnp.sum((S - mean) * (S - mean)) / nf
    inv_std = 1.0 / (jnp.sqrt(var) + eps)

    lo0 = jnp.min(S)
    hi0 = jnp.max(S) + 1.0

    def vbody(_, c):
        lo, hi = c
        mid = 0.5 * (lo + hi)
        ge = jnp.sum((S >= mid).astype(jnp.float32)) >= kf
        return (jnp.where(ge, mid, lo), jnp.where(ge, hi, mid))

    v, _ = lax.fori_loop(0, 44, vbody, (lo0, hi0))

    c_gt = jnp.sum((S > v).astype(jnp.float32))
    need = kf - c_gt

    gix = lax.broadcasted_iota(jnp.int32, (1, N_NODES), 1).astype(jnp.float32)
    iseq = (S == v)

    def ibody(_, c):
        lo, hi = c
        mid = jnp.floor(0.5 * (lo + hi))
        ge = jnp.sum(jnp.where(iseq & (gix <= mid), 1.0, 0.0)) >= need
        return (jnp.where(ge, lo, mid), jnp.where(ge, mid, hi))

    _, tid = lax.fori_loop(0, 17, ibody, (-1.0, nf - 1.0))

    sig = jax.nn.sigmoid((S - mean) * inv_std)
    m = (S > v) | (iseq & (gix <= tid))
    contrib = jnp.where(m, jnp.log(sig + eps), jnp.log(1.0 - sig + eps))
    loss_b = -jnp.sum(contrib) / nf

    @pl.when(b == 0)
    def _():
        loss_ref[...] = jnp.zeros((1, 1), jnp.float32)
    loss_ref[...] += jnp.full((1, 1), 1.0 / BATCH) * loss_b

    params_ref[0] = jnp.concatenate(
        [jnp.full((1, 16), mean), jnp.full((1, 16), inv_std),
         jnp.full((1, 16), v), jnp.full((1, 16), tid)], axis=1)


def _sc_pool_kernel(x_hbm, scores_hbm, params_hbm, out_hbm,
                    svmem, pvmem, idxbuf, coefbuf, rowsbuf, accbuf, dmasem):
    c = lax.axis_index("c")
    s = lax.axis_index("s")
    wid = c * 16 + s
    b = wid // WPB
    r = wid % WPB
    rowbase = b * N_NODES + r * W_ROWS

    pltpu.sync_copy(scores_hbm.at[pl.ds(rowbase, W_ROWS)], svmem)
    pltpu.sync_copy(params_hbm.at[pl.ds(b * 64, 64)], pvmem)
    mean = pvmem[pl.ds(0, 16)]
    inv_std = pvmem[pl.ds(16, 16)]
    v = pvmem[pl.ds(32, 16)]
    tid = pvmem[pl.ds(48, 16)]

    zeros16f = jnp.zeros((16,), jnp.float32)
    zeros16i = jnp.zeros((16,), jnp.int32)

    def zbody(i, _):
        idxbuf[pl.ds(i * 16, 16)] = zeros16i
        coefbuf[pl.ds(i * 16, 16)] = zeros16f
        return 0

    lax.fori_loop(0, CAP // 16, zbody, 0)

    iota_i = lax.iota(jnp.int32, 16)
    iota_f = iota_i.astype(jnp.float32)
    inv_k = jnp.full((16,), 1.0 / K, jnp.float32)
    ones16f = jnp.ones((16,), jnp.float32)
    nlimit = jnp.full((16,), float(N_NODES), jnp.float32)

    def scan_body(i, cnt):
        s16 = svmem[pl.ds(i * 16, 16)]
        z = (s16 - mean) * inv_std
        sig = ones16f / (ones16f + jnp.exp(-z))
        gixf = jnp.full((16,), r * W_ROWS + i * 16, jnp.float32) + iota_f
        m = ((s16 > v) | ((s16 == v) & (gixf <= tid))) & (gixf < nlimit)
        coef16 = jnp.where(m, sig, zeros16f) * inv_k
        grow = jnp.full((16,), rowbase + i * 16, jnp.int32) + iota_i
        plsc.store_compressed(idxbuf.at[pl.ds(cnt, 16)], grow, mask=m)
        plsc.store_compressed(coefbuf.at[pl.ds(cnt, 16)], coef16, mask=m)
        return cnt + jnp.sum(m.astype(jnp.int32))

    cnt = lax.fori_loop(0, W_VB, scan_body, jnp.int32(0))

    nch = (cnt + (CHUNK - 1)) // CHUNK
    acc0 = tuple(jnp.zeros((16,), jnp.float32) for _ in range(DIM // 16))

    def gather_body(ch, acc):
        pltpu.async_copy(
            x_hbm.at[idxbuf.at[pl.ds(ch * CHUNK, CHUNK)]],
            rowsbuf, dmasem).wait()

        def row_body(j, acc_in):
            cb = plsc.load_gather(
                coefbuf, [jnp.full((16,), ch * CHUNK + j, jnp.int32)])
            return tuple(
                acc_in[k] + cb * rowsbuf[j, pl.ds(k * 16, 16)]
                for k in range(DIM // 16))

        return lax.fori_loop(0, CHUNK, row_body, acc, unroll=4)

    acc = lax.fori_loop(0, nch, gather_body, acc0)

    for k in range(DIM // 16):
        accbuf[0, pl.ds(k * 16, 16)] = acc[k]

    pltpu.sync_copy(accbuf, out_hbm.at[pl.ds(wid, 1)])


def _reduce_kernel(p_ref, out_ref):
    out_ref[...] = jnp.sum(p_ref[...], axis=1)


@jax.jit
def kernel(x_batch, W):
    w_row = W.reshape(1, DIM)

    n_p1 = (BATCH * N_NODES) // P1_BLK
    scores_l = pl.pallas_call(
        _scores_kernel,
        grid=(n_p1,),
        in_specs=[
            pl.BlockSpec((P1_BLK, DIM), lambda i: (i, 0)),
            pl.BlockSpec((1, DIM), lambda i: (0, 0)),
        ],
        out_specs=pl.BlockSpec((1, 1, P1_BLK), lambda i: (i, 0, 0)),
        out_shape=jax.ShapeDtypeStruct((n_p1, 1, P1_BLK), jnp.float32),
    )(x_batch, w_row)

    s_full = scores_l.reshape(BATCH, 1, N_NODES)

    params, loss = pl.pallas_call(
        _stats_kernel,
        grid=(BATCH,),
        in_specs=[pl.BlockSpec((1, 1, N_NODES), lambda b: (b, 0, 0))],
        out_specs=[
            pl.BlockSpec((1, 1, 64), lambda b: (b, 0, 0)),
            pl.BlockSpec((1, 1), lambda b: (0, 0)),
        ],
        out_shape=[
            jax.ShapeDtypeStruct((BATCH, 1, 64), jnp.float32),
            jax.ShapeDtypeStruct((1, 1), jnp.float32),
        ],
    )(s_full)

    scores_flat = jnp.concatenate(
        [scores_l.reshape(BATCH * N_NODES),
         jnp.zeros((SCORES_PAD,), jnp.float32)])
    params_flat = params.reshape(BATCH * 64)

    mesh = plsc.VectorSubcoreMesh(core_axis_name="c", subcore_axis_name="s",
                                  num_cores=2, num_subcores=16)
    partials = pl.kernel(
        _sc_pool_kernel,
        out_type=jax.ShapeDtypeStruct((32, DIM), jnp.float32),
        mesh=mesh,
        compiler_params=pltpu.CompilerParams(needs_layout_passes=False),
        scratch_types=[
            pltpu.VMEM((W_ROWS,), jnp.float32),
            pltpu.VMEM((64,), jnp.float32),
            pltpu.VMEM((CAP,), jnp.int32),
            pltpu.VMEM((CAP,), jnp.float32),
            pltpu.VMEM((CHUNK, DIM), jnp.float32),
            pltpu.VMEM((1, DIM), jnp.float32),
            pltpu.SemaphoreType.DMA,
        ],
    )(x_batch, scores_flat, params_flat)

    pooled = pl.pallas_call(
        _reduce_kernel,
        in_specs=[pl.BlockSpec((BATCH, WPB, DIM), lambda: (0, 0, 0))],
        out_specs=pl.BlockSpec((BATCH, DIM), lambda: (0, 0)),
        out_shape=jax.ShapeDtypeStruct((BATCH, DIM), jnp.float32),
    )(partials.reshape(BATCH, WPB, DIM))

    return pooled, loss[0, 0]


# speedup vs baseline: 1.2699x; 1.2699x over previous
import functools

import jax
import jax.numpy as jnp
from jax import lax
from jax.experimental import pallas as pl
from jax.experimental.pallas import tpu as pltpu
from jax.experimental.pallas import tpu_sc as plsc

N_NODES = 50000
DIM = 128
BATCH = 4
K = max(1, int(N_NODES * 0.1))
ROW_BLK = 10000
NBLK = N_NODES // ROW_BLK

WPB = 8
W_ROWS = 6256
W_VB = W_ROWS // 16
SCORES_PAD = 64
CAP = 5120
CHUNK = 256

P1_BLK = 25000


def _scores_kernel(x_ref, w_ref, s_ref):
    prod = x_ref[...] * w_ref[...]
    s_ref[0] = jnp.sum(prod, axis=1).reshape(1, P1_BLK)


def _stats_kernel(s_full_ref, params_ref, loss_ref):
    b = pl.program_id(0)
    nf = float(N_NODES)
    kf = float(K)
    eps = 1e-8
    ncol = N_NODES // 8

    S = s_full_ref[0]
    mean = jnp.sum(S) / nf
    var = jnp.sum((S - mean) * (S - mean)) / nf
    inv_std = 1.0 / (jnp.sqrt(var) + eps)

    lo0 = jnp.min(S)
    hi0 = jnp.max(S) + 1.0

    def vbody(_, c):
        lo, hi = c
        mid = 0.5 * (lo + hi)
        ge = jnp.sum((S >= mid).astype(jnp.float32)) >= kf
        return (jnp.where(ge, mid, lo), jnp.where(ge, hi, mid))

    v, _ = lax.fori_loop(0, 44, vbody, (lo0, hi0))

    c_gt = jnp.sum((S > v).astype(jnp.float32))
    need = kf - c_gt

    gix = (lax.broadcasted_iota(jnp.int32, (8, ncol), 0) * ncol
           + lax.broadcasted_iota(jnp.int32, (8, ncol), 1)
           ).astype(jnp.float32)
    iseq = (S == v)

    def ibody(_, c):
        lo, hi = c
        mid = jnp.floor(0.5 * (lo + hi))
        ge = jnp.sum(jnp.where(iseq & (gix <= mid), 1.0, 0.0)) >= need
        return (jnp.where(ge, lo, mid), jnp.where(ge, mid, hi))

    _, tid = lax.fori_loop(0, 17, ibody, (-1.0, nf - 1.0))

    sig = jax.nn.sigmoid((S - mean) * inv_std)
    m = (S > v) | (iseq & (gix <= tid))
    contrib = jnp.where(m, jnp.log(sig + eps), jnp.log(1.0 - sig + eps))
    loss_b = -jnp.sum(contrib) / nf

    @pl.when(b == 0)
    def _():
        loss_ref[...] = jnp.zeros((1, 1), jnp.float32)
    loss_ref[...] += jnp.full((1, 1), 1.0 / BATCH) * loss_b

    params_ref[0] = jnp.concatenate(
        [jnp.full((1, 16), mean), jnp.full((1, 16), inv_std),
         jnp.full((1, 16), v), jnp.full((1, 16), tid)], axis=1)


def _sc_pool_kernel(x_hbm, scores_hbm, params_hbm, out_hbm,
                    svmem, pvmem, idxbuf, coefbuf, rowsbuf, accbuf, dmasem):
    c = lax.axis_index("c")
    s = lax.axis_index("s")
    wid = c * 16 + s
    b = wid // WPB
    r = wid % WPB
    rowbase = b * N_NODES + r * W_ROWS

    pltpu.sync_copy(scores_hbm.at[pl.ds(rowbase, W_ROWS)], svmem)
    pltpu.sync_copy(params_hbm.at[pl.ds(b * 64, 64)], pvmem)
    mean = pvmem[pl.ds(0, 16)]
    inv_std = pvmem[pl.ds(16, 16)]
    v = pvmem[pl.ds(32, 16)]
    tid = pvmem[pl.ds(48, 16)]

    zeros16f = jnp.zeros((16,), jnp.float32)
    zeros16i = jnp.zeros((16,), jnp.int32)

    def zbody(i, _):
        idxbuf[pl.ds(i * 16, 16)] = zeros16i
        coefbuf[pl.ds(i * 16, 16)] = zeros16f
        return 0

    lax.fori_loop(0, CAP // 16, zbody, 0)

    iota_i = lax.iota(jnp.int32, 16)
    iota_f = iota_i.astype(jnp.float32)
    inv_k = jnp.full((16,), 1.0 / K, jnp.float32)
    ones16f = jnp.ones((16,), jnp.float32)
    nlimit = jnp.full((16,), float(N_NODES), jnp.float32)

    def scan_body(i, cnt):
        s16 = svmem[pl.ds(i * 16, 16)]
        z = (s16 - mean) * inv_std
        sig = ones16f / (ones16f + jnp.exp(-z))
        gixf = jnp.full((16,), r * W_ROWS + i * 16, jnp.float32) + iota_f
        m = ((s16 > v) | ((s16 == v) & (gixf <= tid))) & (gixf < nlimit)
        coef16 = jnp.where(m, sig, zeros16f) * inv_k
        grow = jnp.full((16,), rowbase + i * 16, jnp.int32) + iota_i
        plsc.store_compressed(idxbuf.at[pl.ds(cnt, 16)], grow, mask=m)
        plsc.store_compressed(coefbuf.at[pl.ds(cnt, 16)], coef16, mask=m)
        return cnt + jnp.sum(m.astype(jnp.int32))

    cnt = lax.fori_loop(0, W_VB, scan_body, jnp.int32(0))

    nch = (cnt + (CHUNK - 1)) // CHUNK
    acc0 = tuple(jnp.zeros((16,), jnp.float32) for _ in range(DIM // 16))

    def gather_body(ch, acc):
        pltpu.async_copy(
            x_hbm.at[idxbuf.at[pl.ds(ch * CHUNK, CHUNK)]],
            rowsbuf, dmasem).wait()

        def row_body(j, acc_in):
            cb = plsc.load_gather(
                coefbuf, [jnp.full((16,), ch * CHUNK + j, jnp.int32)])
            return tuple(
                acc_in[k] + cb * rowsbuf[j, pl.ds(k * 16, 16)]
                for k in range(DIM // 16))

        return lax.fori_loop(0, CHUNK, row_body, acc)

    acc = lax.fori_loop(0, nch, gather_body, acc0)

    for k in range(DIM // 16):
        accbuf[0, pl.ds(k * 16, 16)] = acc[k]

    pltpu.sync_copy(accbuf, out_hbm.at[pl.ds(wid, 1)])


def _reduce_kernel(p_ref, out_ref):
    out_ref[...] = jnp.sum(p_ref[...], axis=1)


@jax.jit
def kernel(x_batch, W):
    w_row = W.reshape(1, DIM)

    n_p1 = (BATCH * N_NODES) // P1_BLK
    scores_l = pl.pallas_call(
        _scores_kernel,
        grid=(n_p1,),
        in_specs=[
            pl.BlockSpec((P1_BLK, DIM), lambda i: (i, 0)),
            pl.BlockSpec((1, DIM), lambda i: (0, 0)),
        ],
        out_specs=pl.BlockSpec((1, 1, P1_BLK), lambda i: (i, 0, 0)),
        out_shape=jax.ShapeDtypeStruct((n_p1, 1, P1_BLK), jnp.float32),
    )(x_batch, w_row)

    s_full = scores_l.reshape(BATCH, 8, N_NODES // 8)

    params, loss = pl.pallas_call(
        _stats_kernel,
        grid=(BATCH,),
        in_specs=[pl.BlockSpec((1, 8, N_NODES // 8), lambda b: (b, 0, 0))],
        out_specs=[
            pl.BlockSpec((1, 1, 64), lambda b: (b, 0, 0)),
            pl.BlockSpec((1, 1), lambda b: (0, 0)),
        ],
        out_shape=[
            jax.ShapeDtypeStruct((BATCH, 1, 64), jnp.float32),
            jax.ShapeDtypeStruct((1, 1), jnp.float32),
        ],
    )(s_full)

    scores_flat = jnp.concatenate(
        [scores_l.reshape(BATCH * N_NODES),
         jnp.zeros((SCORES_PAD,), jnp.float32)])
    params_flat = params.reshape(BATCH * 64)

    mesh = plsc.VectorSubcoreMesh(core_axis_name="c", subcore_axis_name="s",
                                  num_cores=2, num_subcores=16)
    partials = pl.kernel(
        _sc_pool_kernel,
        out_type=jax.ShapeDtypeStruct((32, DIM), jnp.float32),
        mesh=mesh,
        compiler_params=pltpu.CompilerParams(needs_layout_passes=False),
        scratch_types=[
            pltpu.VMEM((W_ROWS,), jnp.float32),
            pltpu.VMEM((64,), jnp.float32),
            pltpu.VMEM((CAP,), jnp.int32),
            pltpu.VMEM((CAP,), jnp.float32),
            pltpu.VMEM((CHUNK, DIM), jnp.float32),
            pltpu.VMEM((1, DIM), jnp.float32),
            pltpu.SemaphoreType.DMA,
        ],
    )(x_batch, scores_flat, params_flat)

    pooled = pl.pallas_call(
        _reduce_kernel,
        in_specs=[pl.BlockSpec((BATCH, WPB, DIM), lambda: (0, 0, 0))],
        out_specs=pl.BlockSpec((BATCH, DIM), lambda: (0, 0)),
        out_shape=jax.ShapeDtypeStruct((BATCH, DIM), jnp.float32),
    )(partials.reshape(BATCH, WPB, DIM))

    return pooled, loss[0, 0]
